# baseline (speedup 1.0000x reference)
import functools

import jax
import jax.numpy as jnp
from jax import lax
from jax.experimental import pallas as pl
from jax.experimental.pallas import tpu as pltpu
from jax.experimental.pallas import tpu_sc as plsc

B = 16384
D = 128
NC, NS = 2, 16
NW = NC * NS
BPW = B // NW
CH = 64
NCK = BPW // CH
L = 16
MB = 8
NMST = 3 * NCK


def _sc_gather(u3, p3, n3, wf1r, umf, imf, umlp, imlp):
    mesh = plsc.VectorSubcoreMesh(core_axis_name="c", subcore_axis_name="s")

    @functools.partial(
        pl.kernel,
        mesh=mesh,
        out_type=[
            jax.ShapeDtypeStruct((B, D), jnp.float32),
            jax.ShapeDtypeStruct((B, D), jnp.float32),
            jax.ShapeDtypeStruct((B, D), jnp.float32),
            jax.ShapeDtypeStruct((B,), jnp.float32),
            jax.ShapeDtypeStruct((B,), jnp.float32),
        ],
        scratch_types=[
            pltpu.VMEM((NCK, CH), jnp.int32),
            pltpu.VMEM((NCK, CH), jnp.int32),
            pltpu.VMEM((NCK, CH), jnp.int32),
            pltpu.VMEM((1, D), jnp.float32),
            pltpu.VMEM((CH, D), jnp.float32),
            pltpu.VMEM((CH, D), jnp.float32),
            pltpu.VMEM((CH, D), jnp.float32),
            pltpu.VMEM((CH, D), jnp.float32),
            pltpu.VMEM((CH, D), jnp.float32),
            pltpu.VMEM((CH, D), jnp.float32),
        ] + [pltpu.VMEM((CH, D), jnp.float32)] * MB + [
            pltpu.VMEM((BPW + L,), jnp.float32),
            pltpu.VMEM((BPW + L,), jnp.float32),
            pltpu.VMEM((4 * L,), jnp.float32),
            pltpu.VMEM((4 * L,), jnp.float32),
        ] + [pltpu.SemaphoreType.DMA] * (7 + 2 * MB),
    )
    def k(u_idx_h, p_idx_h, n_idx_h, wf1_h, umf_h, imf_h, umlp_h, imlp_h,
          o_umlp, o_imlp_p, o_imlp_n, o_sp, o_sn,
          xu, xp, xn, wv, ub0, ub1, pb0, pb1, nb0, nb1,
          *rest):
        gbs = rest[:MB]
        spb, snb, t32p, t32n = rest[MB:MB + 4]
        su0, su1, sp0, sp1, sn0, sn1 = rest[MB + 4:MB + 10]
        sgs = rest[MB + 10:MB + 10 + MB]
        sws = rest[MB + 10 + MB:MB + 10 + 2 * MB]
        si = rest[MB + 10 + 2 * MB]
        wid = lax.axis_index("s") * NC + lax.axis_index("c")
        ih = (pltpu.async_copy(u_idx_h.at[wid], xu, si),
              pltpu.async_copy(p_idx_h.at[wid], xp, si),
              pltpu.async_copy(n_idx_h.at[wid], xn, si),
              pltpu.async_copy(wf1_h, wv, si))
        for h in ih:
            h.wait()
        base = wid * BPW
        ubs, pbs, nbs = (ub0, ub1), (pb0, pb1), (nb0, nb1)
        sus, sps, sns = (su0, su1), (sp0, sp1), (sn0, sn1)

        def start_mf(c):
            b = c % 2
            return (
                pltpu.async_copy(umf_h.at[xu.at[c]], ubs[b], sus[b]),
                pltpu.async_copy(imf_h.at[xp.at[c]], pbs[b], sps[b]),
                pltpu.async_copy(imf_h.at[xn.at[c]], nbs[b], sns[b]),
            )

        NK = D // L
        z = jnp.zeros((L,), jnp.float32)
        t32p[pl.ds(L, L)] = z
        t32p[pl.ds(3 * L, L)] = z
        t32n[pl.ds(L, L)] = z
        t32n[pl.ds(3 * L, L)] = z

        def compute_chunk(c):
            b = c % 2
            ub, pb, nb = ubs[b], pbs[b], nbs[b]
            wks = [wv[0, pl.ds(k * L, L)] for k in range(NK)]

            def pair_body(rr, _):
                r = rr * 2
                vs = []
                for dr in (0, 1):
                    part_p = z
                    part_n = z
                    for k in range(NK):
                        t = ub[r + dr, pl.ds(k * L, L)] * wks[k]
                        part_p = part_p + t * pb[r + dr, pl.ds(k * L, L)]
                        part_n = part_n + t * nb[r + dr, pl.ds(k * L, L)]
                    vs += [part_p, part_n]
                bufs4 = (t32p, t32n, t32p, t32n)
                offs = (0, 0, 2 * L, 2 * L)
                for sh in (8, 4, 2, 1):
                    for i in range(4):
                        bufs4[i][pl.ds(offs[i], L)] = vs[i]
                    for i in range(4):
                        vs[i] = vs[i] + bufs4[i][pl.ds(offs[i] + sh, L)]
                spb[pl.ds(c * CH + r, L)] = vs[0]
                snb[pl.ds(c * CH + r, L)] = vs[1]
                spb[pl.ds(c * CH + r + 1, L)] = vs[2]
                snb[pl.ds(c * CH + r + 1, L)] = vs[3]
                return 0

            lax.fori_loop(0, CH // 2, pair_body, 0)

        steps = []
        for c in range(NCK):
            for tbl, xi, out in ((umlp_h, xu, o_umlp),
                                 (imlp_h, xp, o_imlp_p),
                                 (imlp_h, xn, o_imlp_n)):
                steps.append((tbl, xi, c, out))

        def start_mlp_gather(j):
            tbl, xi, c, _ = steps[j]
            return pltpu.async_copy(tbl.at[xi.at[c]], gbs[j % MB], sgs[j % MB])

        def start_mlp_wb(j):
            _, _, c, out = steps[j]
            return pltpu.async_copy(
                gbs[j % MB], out.at[pl.ds(base + c * CH, CH)], sws[j % MB])

        mfg = {0: start_mf(0), 1: start_mf(1)}
        glh = {j: start_mlp_gather(j) for j in range(6)}
        wbh = {}
        mstep = 0
        for c in range(NCK):
            for h in mfg[c]:
                h.wait()
            compute_chunk(c)
            if c + 2 < NCK:
                mfg[c + 2] = start_mf(c + 2)
            for _ in range(3):
                j = mstep
                mstep += 1
                glh[j].wait()
                wbh[j] = start_mlp_wb(j)
                nxt = j + 6
                if nxt < NMST:
                    if j >= 2:
                        wbh[j - 2].wait()
                    glh[nxt] = start_mlp_gather(nxt)
        hsp = pltpu.async_copy(
            spb.at[pl.ds(0, BPW)], o_sp.at[pl.ds(base, BPW)], si)
        hsn = pltpu.async_copy(
            snb.at[pl.ds(0, BPW)], o_sn.at[pl.ds(base, BPW)], si)
        for j in range(NMST - 8, NMST):
            wbh[j].wait()
        hsp.wait()
        hsn.wait()

    return k(u3, p3, n3, wf1r, umf, imf, umlp, imlp)


def _dense(umlp_g, imlp_p_g, imlp_n_g, smf_p, smf_n,
           w1a, w1b, b1r, wf2, bfv):
    BB = 4096

    def body(umlp_r, imlp_p_r, imlp_n_r, sp_r, sn_r,
             w1a_r, w1b_r, b1_r, wf2_r, bf_r, pos_r, neg_r):
        hu = jnp.dot(umlp_r[...], w1a_r[...],
                     preferred_element_type=jnp.float32)
        hp = jnp.maximum(
            hu + jnp.dot(imlp_p_r[...], w1b_r[...],
                         preferred_element_type=jnp.float32) + b1_r[...], 0.0)
        hn = jnp.maximum(
            hu + jnp.dot(imlp_n_r[...], w1b_r[...],
                         preferred_element_type=jnp.float32) + b1_r[...], 0.0)
        bf0 = bf_r[0, 0]
        pos_r[...] = (sp_r[...].reshape(BB, 1)
                      + jnp.sum(hp * wf2_r[...], axis=1, keepdims=True) + bf0)
        neg_r[...] = (sn_r[...].reshape(BB, 1)
                      + jnp.sum(hn * wf2_r[...], axis=1, keepdims=True) + bf0)

    bspec_in = pl.BlockSpec((BB, D), lambda i: (i, 0))
    bspec_s = pl.BlockSpec((1, BB), lambda i: (0, i))
    bspec_w = pl.BlockSpec((D, D), lambda i: (0, 0))
    bspec_r = pl.BlockSpec((1, D), lambda i: (0, 0))
    bspec_bf = pl.BlockSpec((1, 1), lambda i: (0, 0))
    bspec_out = pl.BlockSpec((BB, 1), lambda i: (i, 0))
    return pl.pallas_call(
        body,
        grid=(B // BB,),
        in_specs=[bspec_in] * 3 + [bspec_s, bspec_s]
        + [bspec_w, bspec_w, bspec_r, bspec_r, bspec_bf],
        out_specs=[bspec_out, bspec_out],
        out_shape=[jax.ShapeDtypeStruct((B, 1), jnp.float32)] * 2,
    )(umlp_g, imlp_p_g, imlp_n_g, smf_p, smf_n,
      w1a, w1b, b1r, wf2, bfv)


def kernel(batch_user, batch_pos_item, batch_neg_item,
           user_emb_MF, item_emb_MF, user_emb_MLP, item_emb_MLP,
           W1, b1, Wf, bf):
    u3 = batch_user.astype(jnp.int32).reshape(NW, NCK, CH)
    p3 = batch_pos_item.astype(jnp.int32).reshape(NW, NCK, CH)
    n3 = batch_neg_item.astype(jnp.int32).reshape(NW, NCK, CH)
    wf1r = Wf[:D, 0].reshape(1, D)
    o_umlp, o_imlp_p, o_imlp_n, o_sp, o_sn = _sc_gather(
        u3, p3, n3, wf1r,
        user_emb_MF, item_emb_MF,
        user_emb_MLP, item_emb_MLP)
    w1a = W1[:D]
    w1b = W1[D:]
    b1r = b1.reshape(1, D)
    wf2 = Wf[D:, 0].reshape(1, D)
    bfv = bf.reshape(1, 1)
    pos, neg = _dense(o_umlp, o_imlp_p, o_imlp_n,
                      o_sp.reshape(1, B), o_sn.reshape(1, B),
                      w1a, w1b, b1r, wf2, bfv)
    return (pos, neg)


# speedup vs baseline: 1.2550x; 1.2550x over previous
import functools

import jax
import jax.numpy as jnp
from jax import lax
from jax.experimental import pallas as pl
from jax.experimental.pallas import tpu as pltpu
from jax.experimental.pallas import tpu_sc as plsc

B = 16384
D = 128
NC, NS = 2, 16
NW = NC * NS
BPW = B // NW
CH = 64
NCK = BPW // CH
L = 16
MB = 8
NMST = 3 * NCK


def _sc_gather(u3, p3, n3, wf1r, umf, imf, umlp, imlp):
    mesh = plsc.VectorSubcoreMesh(core_axis_name="c", subcore_axis_name="s")

    @functools.partial(
        pl.kernel,
        mesh=mesh,
        out_type=[
            jax.ShapeDtypeStruct((B, D), jnp.float32),
            jax.ShapeDtypeStruct((B, D), jnp.float32),
            jax.ShapeDtypeStruct((B, D), jnp.float32),
            jax.ShapeDtypeStruct((B,), jnp.float32),
            jax.ShapeDtypeStruct((B,), jnp.float32),
        ],
        scratch_types=[
            pltpu.VMEM((NCK, CH), jnp.int32),
            pltpu.VMEM((NCK, CH), jnp.int32),
            pltpu.VMEM((NCK, CH), jnp.int32),
            pltpu.VMEM((1, D), jnp.float32),
            pltpu.VMEM((CH, D), jnp.float32),
            pltpu.VMEM((CH, D), jnp.float32),
            pltpu.VMEM((CH, D), jnp.float32),
            pltpu.VMEM((CH, D), jnp.float32),
            pltpu.VMEM((CH, D), jnp.float32),
            pltpu.VMEM((CH, D), jnp.float32),
        ] + [pltpu.VMEM((CH, D), jnp.float32)] * MB + [
            pltpu.VMEM((BPW + L,), jnp.float32),
            pltpu.VMEM((BPW + L,), jnp.float32),
            pltpu.VMEM((4 * L,), jnp.float32),
            pltpu.VMEM((4 * L,), jnp.float32),
        ] + [pltpu.SemaphoreType.DMA] * (7 + 2 * MB),
    )
    def k(u_idx_h, p_idx_h, n_idx_h, wf1_h, umf_h, imf_h, umlp_h, imlp_h,
          o_umlp, o_imlp_p, o_imlp_n, o_sp, o_sn,
          xu, xp, xn, wv, ub0, ub1, pb0, pb1, nb0, nb1,
          *rest):
        gbs = rest[:MB]
        spb, snb, t32p, t32n = rest[MB:MB + 4]
        su0, su1, sp0, sp1, sn0, sn1 = rest[MB + 4:MB + 10]
        sgs = rest[MB + 10:MB + 10 + MB]
        sws = rest[MB + 10 + MB:MB + 10 + 2 * MB]
        si = rest[MB + 10 + 2 * MB]
        wid = lax.axis_index("s") * NC + lax.axis_index("c")
        ih = (pltpu.async_copy(u_idx_h.at[wid], xu, si),
              pltpu.async_copy(p_idx_h.at[wid], xp, si),
              pltpu.async_copy(n_idx_h.at[wid], xn, si),
              pltpu.async_copy(wf1_h, wv, si))
        for h in ih:
            h.wait()
        base = wid * BPW
        ubs, pbs, nbs = (ub0, ub1), (pb0, pb1), (nb0, nb1)
        sus, sps, sns = (su0, su1), (sp0, sp1), (sn0, sn1)

        def start_mf(c):
            b = c % 2
            return (
                pltpu.async_copy(umf_h.at[xu.at[c]], ubs[b], sus[b]),
                pltpu.async_copy(imf_h.at[xp.at[c]], pbs[b], sps[b]),
                pltpu.async_copy(imf_h.at[xn.at[c]], nbs[b], sns[b]),
            )

        NK = D // L
        z = jnp.zeros((L,), jnp.float32)
        t32p[pl.ds(L, L)] = z
        t32p[pl.ds(3 * L, L)] = z
        t32n[pl.ds(L, L)] = z
        t32n[pl.ds(3 * L, L)] = z

        def compute_chunk(c):
            b = c % 2
            ub, pb, nb = ubs[b], pbs[b], nbs[b]
            wks = [wv[0, pl.ds(k * L, L)] for k in range(NK)]

            def pair_body(rr, _):
                r = rr * 2
                vs = []
                for dr in (0, 1):
                    part_p = z
                    part_n = z
                    for k in range(NK):
                        t = ub[r + dr, pl.ds(k * L, L)] * wks[k]
                        part_p = part_p + t * pb[r + dr, pl.ds(k * L, L)]
                        part_n = part_n + t * nb[r + dr, pl.ds(k * L, L)]
                    vs += [part_p, part_n]
                bufs4 = (t32p, t32n, t32p, t32n)
                offs = (0, 0, 2 * L, 2 * L)
                for sh in (8, 4, 2, 1):
                    for i in range(4):
                        bufs4[i][pl.ds(offs[i], L)] = vs[i]
                    for i in range(4):
                        vs[i] = vs[i] + bufs4[i][pl.ds(offs[i] + sh, L)]
                spb[pl.ds(c * CH + r, L)] = vs[0]
                snb[pl.ds(c * CH + r, L)] = vs[1]
                spb[pl.ds(c * CH + r + 1, L)] = vs[2]
                snb[pl.ds(c * CH + r + 1, L)] = vs[3]
                return 0

            lax.fori_loop(0, CH // 2, pair_body, 0)

        steps = []
        for c in range(NCK):
            for tbl, xi, out in ((umlp_h, xu, o_umlp),
                                 (imlp_h, xp, o_imlp_p),
                                 (imlp_h, xn, o_imlp_n)):
                steps.append((tbl, xi, c, out))

        def start_mlp_gather(j):
            tbl, xi, c, _ = steps[j]
            return pltpu.async_copy(tbl.at[xi.at[c]], gbs[j % MB], sgs[j % MB])

        def start_mlp_wb(j):
            _, _, c, out = steps[j]
            return pltpu.async_copy(
                gbs[j % MB], out.at[pl.ds(base + c * CH, CH)], sws[j % MB])

        mfg = {0: start_mf(0), 1: start_mf(1)}
        glh = {j: start_mlp_gather(j) for j in range(6)}
        wbh = {}
        mstep = 0
        for c in range(NCK):
            for h in mfg[c]:
                h.wait()
            compute_chunk(c)
            if c + 2 < NCK:
                mfg[c + 2] = start_mf(c + 2)
            for _ in range(3):
                j = mstep
                mstep += 1
                glh[j].wait()
                wbh[j] = start_mlp_wb(j)
                nxt = j + 6
                if nxt < NMST:
                    if j >= 2:
                        wbh[j - 2].wait()
                    glh[nxt] = start_mlp_gather(nxt)
        hsp = pltpu.async_copy(
            spb.at[pl.ds(0, BPW)], o_sp.at[pl.ds(base, BPW)], si)
        hsn = pltpu.async_copy(
            snb.at[pl.ds(0, BPW)], o_sn.at[pl.ds(base, BPW)], si)
        for j in range(NMST - 8, NMST):
            wbh[j].wait()
        hsp.wait()
        hsn.wait()

    return k(u3, p3, n3, wf1r, umf, imf, umlp, imlp)


def _dense(umlp_g, imlp_p_g, imlp_n_g, smf_p, smf_n,
           w1a, w1b, b1r, wf2, bfv):
    BB = 4096

    def body(umlp_r, imlp_p_r, imlp_n_r, sp_r, sn_r,
             w1a_r, w1b_r, b1_r, wf2_r, bf_r, pos_r, neg_r):
        hu = jnp.dot(umlp_r[...], w1a_r[...],
                     preferred_element_type=jnp.float32)
        hp = jnp.maximum(
            hu + jnp.dot(imlp_p_r[...], w1b_r[...],
                         preferred_element_type=jnp.float32) + b1_r[...], 0.0)
        hn = jnp.maximum(
            hu + jnp.dot(imlp_n_r[...], w1b_r[...],
                         preferred_element_type=jnp.float32) + b1_r[...], 0.0)
        bf0 = bf_r[0, 0]
        dn = (((1,), (1,)), ((), ()))
        mlp_p = lax.dot_general(wf2_r[...], hp, dn,
                                preferred_element_type=jnp.float32)
        mlp_n = lax.dot_general(wf2_r[...], hn, dn,
                                preferred_element_type=jnp.float32)
        pos_r[...] = sp_r[...] + mlp_p + bf0
        neg_r[...] = sn_r[...] + mlp_n + bf0

    bspec_in = pl.BlockSpec((BB, D), lambda i: (i, 0))
    bspec_s = pl.BlockSpec((1, BB), lambda i: (0, i))
    bspec_w = pl.BlockSpec((D, D), lambda i: (0, 0))
    bspec_r = pl.BlockSpec((1, D), lambda i: (0, 0))
    bspec_bf = pl.BlockSpec((1, 1), lambda i: (0, 0))
    bspec_out = pl.BlockSpec((1, BB), lambda i: (0, i))
    return pl.pallas_call(
        body,
        grid=(B // BB,),
        in_specs=[bspec_in] * 3 + [bspec_s, bspec_s]
        + [bspec_w, bspec_w, bspec_r, bspec_r, bspec_bf],
        out_specs=[bspec_out, bspec_out],
        out_shape=[jax.ShapeDtypeStruct((1, B), jnp.float32)] * 2,
    )(umlp_g, imlp_p_g, imlp_n_g, smf_p, smf_n,
      w1a, w1b, b1r, wf2, bfv)


def kernel(batch_user, batch_pos_item, batch_neg_item,
           user_emb_MF, item_emb_MF, user_emb_MLP, item_emb_MLP,
           W1, b1, Wf, bf):
    u3 = batch_user.astype(jnp.int32).reshape(NW, NCK, CH)
    p3 = batch_pos_item.astype(jnp.int32).reshape(NW, NCK, CH)
    n3 = batch_neg_item.astype(jnp.int32).reshape(NW, NCK, CH)
    wf1r = Wf[:D, 0].reshape(1, D)
    o_umlp, o_imlp_p, o_imlp_n, o_sp, o_sn = _sc_gather(
        u3, p3, n3, wf1r,
        user_emb_MF, item_emb_MF,
        user_emb_MLP, item_emb_MLP)
    w1a = W1[:D]
    w1b = W1[D:]
    b1r = b1.reshape(1, D)
    wf2 = Wf[D:, 0].reshape(1, D)
    bfv = bf.reshape(1, 1)
    pos, neg = _dense(o_umlp, o_imlp_p, o_imlp_n,
                      o_sp.reshape(1, B), o_sn.reshape(1, B),
                      w1a, w1b, b1r, wf2, bfv)
    return (pos.reshape(B, 1), neg.reshape(B, 1))
